# baseline (speedup 1.0000x reference)
import functools

import jax
import jax.numpy as jnp
from jax import lax
from jax.experimental import pallas as pl
from jax.experimental.pallas import tpu as pltpu
from jax.experimental.pallas import tpu_sc as plsc

N_NODES = 10000
N_EDGES = 320000
CLUS = 128
FOU1 = 64

NC = 2
NS = 16
NW = NC * NS
EW = N_EDGES // NW
CH = 80
NCH = EW // CH
NZT = 10
ROWS_PER_TILE = N_NODES // NZT

def _sc_pass1_body(xflat_hbm, src_hbm, dst_hbm, w_hbm, z8_hbm, z4_hbm,
                   stag0_hbm,
                   agg_out, deg_out,
                   src_v, dst_v, wvec_v, wbuf0_v, wbuf1_v, x_v,
                   stag0_v, stag1_v, zb8_v, zb4_v,
                   agg_sh, deg_sh, sem_a0, sem_d0, sem_a1, sem_d1):
    cid = lax.axis_index("c")
    sid = lax.axis_index("s")
    wid = cid * NS + sid

    @pl.when(sid < NZT)
    def _zero():
        zrows = pl.ds(sid * ROWS_PER_TILE, ROWS_PER_TILE)
        pltpu.sync_copy(z8_hbm, zb8_v)
        pltpu.sync_copy(zb8_v, agg_sh.at[zrows])
        pltpu.sync_copy(z4_hbm, zb4_v)
        pltpu.sync_copy(zb4_v, deg_sh.at[zrows])
    pltpu.sync_copy(z4_hbm.at[pl.ds(0, CH)], wbuf0_v)
    pltpu.sync_copy(z4_hbm.at[pl.ds(0, CH)], wbuf1_v)

    pltpu.sync_copy(xflat_hbm, x_v)
    pltpu.sync_copy(src_hbm.at[wid], src_v)
    pltpu.sync_copy(dst_hbm.at[wid], dst_v)
    pltpu.sync_copy(stag0_hbm, stag0_v)
    pltpu.sync_copy(stag0_hbm, stag1_v)
    plsc.subcore_barrier()

    lanes = lax.iota(jnp.int32, 16)
    zeros16 = jnp.zeros((16,), jnp.int32)
    bufs = ((stag0_v, wbuf0_v, sem_a0, sem_d0), (stag1_v, wbuf1_v, sem_a1, sem_d1))

    def fill(k, b):
        stag, wbuf, _, _ = bufs[b]
        pltpu.sync_copy(w_hbm.at[wid, pl.ds(k * CH, CH)], wvec_v)

        def group(g, carry):
            s16 = src_v[k, pl.ds(g * 16, 16)]
            rows = g * 16 + lanes
            for c in range(5):
                xc = plsc.load_gather(x_v, [s16 * 5 + c])
                plsc.store_scatter(stag, [rows, jnp.full((16,), c, jnp.int32)], xc)
            wv = wvec_v[pl.ds(g * 16, 16)]
            plsc.store_scatter(wbuf, [rows, zeros16], wv)
            return carry

        lax.fori_loop(0, CH // 16, group, 0, unroll=True)

    def start(k, b):
        stag, wbuf, sa, sd = bufs[b]
        pltpu.async_copy(stag, agg_sh.at[dst_v.at[k]], sa, add=True)
        pltpu.async_copy(wbuf, deg_sh.at[src_v.at[k]], sd, add=True)

    def wait(k, b):
        stag, wbuf, sa, sd = bufs[b]
        pltpu.make_async_copy(stag, agg_sh.at[dst_v.at[k]], sa).wait()
        pltpu.make_async_copy(wbuf, deg_sh.at[src_v.at[k]], sd).wait()

    fill(0, 0)
    start(0, 0)

    def pair(j, carry):
        k0 = 2 * j
        fill(k0 + 1, 1)
        start(k0 + 1, 1)
        wait(k0, 0)

        @pl.when(j + 1 < NCH // 2)
        def _next():
            fill(k0 + 2, 0)
            start(k0 + 2, 0)

        wait(k0 + 1, 1)
        return carry

    lax.fori_loop(0, NCH // 2, pair, 0, unroll=False)
    if NCH % 2:
        fill(NCH - 1, 0)
        start(NCH - 1, 0)
        wait(NCH - 1, 0)
    plsc.subcore_barrier()

    @pl.when(sid < NZT)
    def _writeout():
        rows = pl.ds(sid * ROWS_PER_TILE, ROWS_PER_TILE)
        pltpu.sync_copy(agg_sh.at[rows], zb8_v)
        pltpu.sync_copy(zb8_v, agg_out.at[cid, rows])
        pltpu.sync_copy(deg_sh.at[rows], zb4_v)
        pltpu.sync_copy(zb4_v, deg_out.at[cid, rows])


def _sc_pass2_body(q_hbm, src_hbm, dst_hbm, w_hbm, part_out,
                   src_v, dst_v, w_v,
                   qa0_v, qb0_v, qa1_v, qb1_v, acc_v,
                   sem_a0, sem_b0, sem_a1, sem_b1):
    cid = lax.axis_index("c")
    sid = lax.axis_index("s")
    wid = cid * NS + sid

    pltpu.sync_copy(src_hbm.at[wid], src_v)
    pltpu.sync_copy(dst_hbm.at[wid], dst_v)
    pltpu.sync_copy(w_hbm.at[wid], w_v)

    bufs = ((qa0_v, qb0_v, sem_a0, sem_b0), (qa1_v, qb1_v, sem_a1, sem_b1))

    def start(k, b):
        qa, qb, sa, sb = bufs[b]
        pltpu.async_copy(q_hbm.at[src_v.at[k]], qa, sa)
        pltpu.async_copy(q_hbm.at[dst_v.at[k]], qb, sb)

    def compute(k, b, acc):
        qa, qb, sa, sb = bufs[b]
        pltpu.make_async_copy(q_hbm.at[src_v.at[k]], qa, sa).wait()
        pltpu.make_async_copy(q_hbm.at[dst_v.at[k]], qb, sb).wait()

        def row(r, acc):
            wbc = plsc.load_gather(w_v, [jnp.full((16,), k * CH + r, jnp.int32)])
            d = qa[r, pl.ds(0, 16)] * qb[r, pl.ds(0, 16)]
            for u in range(1, CLUS // 16):
                d = d + qa[r, pl.ds(u * 16, 16)] * qb[r, pl.ds(u * 16, 16)]
            return acc + wbc * d

        return lax.fori_loop(0, CH, row, acc, unroll=16)

    start(0, 0)

    def pair(j, acc):
        k0 = 2 * j
        start(k0 + 1, 1)
        acc = compute(k0, 0, acc)

        @pl.when(j + 1 < NCH // 2)
        def _next():
            start(k0 + 2, 0)

        return compute(k0 + 1, 1, acc)

    acc = lax.fori_loop(0, NCH // 2, pair, jnp.zeros((16,), jnp.float32),
                        unroll=False)
    if NCH % 2:
        start(NCH - 1, 0)
        acc = compute(NCH - 1, 0, acc)
    acc_v[...] = acc
    pltpu.sync_copy(acc_v, part_out.at[pl.ds(wid * 16, 16)])


@functools.cache
def _sc_kernels():
    mesh = plsc.VectorSubcoreMesh(core_axis_name="c", subcore_axis_name="s")
    params = pltpu.CompilerParams(needs_layout_passes=False, use_tc_tiling_on_sc=False)
    pass1 = pl.kernel(
        _sc_pass1_body,
        mesh=mesh,
        compiler_params=params,
        out_type=[
            jax.ShapeDtypeStruct((NC, N_NODES, 16), jnp.float32),
            jax.ShapeDtypeStruct((NC, N_NODES, 16), jnp.float32),
        ],
        scratch_types=[
            pltpu.VMEM((NCH, CH), jnp.int32),
            pltpu.VMEM((NCH, CH), jnp.int32),
            pltpu.VMEM((CH,), jnp.float32),
            pltpu.VMEM((CH, 16), jnp.float32),
            pltpu.VMEM((CH, 16), jnp.float32),
            pltpu.VMEM((N_NODES * 5,), jnp.float32),
            pltpu.VMEM((CH, 16), jnp.float32),
            pltpu.VMEM((CH, 16), jnp.float32),
            pltpu.VMEM((ROWS_PER_TILE, 16), jnp.float32),
            pltpu.VMEM((ROWS_PER_TILE, 16), jnp.float32),
            pltpu.VMEM_SHARED((N_NODES, 16), jnp.float32),
            pltpu.VMEM_SHARED((N_NODES, 16), jnp.float32),
            pltpu.SemaphoreType.DMA,
            pltpu.SemaphoreType.DMA,
            pltpu.SemaphoreType.DMA,
            pltpu.SemaphoreType.DMA,
        ],
    )
    pass2 = pl.kernel(
        _sc_pass2_body,
        mesh=mesh,
        compiler_params=params,
        out_type=jax.ShapeDtypeStruct((NW * 16,), jnp.float32),
        scratch_types=[
            pltpu.VMEM((NCH, CH), jnp.int32),
            pltpu.VMEM((NCH, CH), jnp.int32),
            pltpu.VMEM((EW,), jnp.float32),
            pltpu.VMEM((CH, CLUS), jnp.float32),
            pltpu.VMEM((CH, CLUS), jnp.float32),
            pltpu.VMEM((CH, CLUS), jnp.float32),
            pltpu.VMEM((CH, CLUS), jnp.float32),
            pltpu.VMEM((16,), jnp.float32),
            pltpu.SemaphoreType.DMA,
            pltpu.SemaphoreType.DMA,
            pltpu.SemaphoreType.DMA,
            pltpu.SemaphoreType.DMA,
        ],
    )
    return pass1, pass2


_RB = 2000
_NB = N_NODES // _RB


def _tc_body(agg_ref, deg_ref, x_ref, wa_ref, wx_ref, bcat_ref,
             w2e_w_ref, w2e_root_ref, w2e_b_ref,
             lin1_w_ref, lin1_b_ref, lin2_w_ref, lin2_b_ref,
             q_ref, sums2_ref, logp_ref,
             out_acc, s2_acc):
    i = pl.program_id(0)

    @pl.when(i == 0)
    def _init():
        out_acc[...] = jnp.zeros_like(out_acc)
        s2_acc[0] = 0.0

    ag = agg_ref[0] + agg_ref[1]
    cnt = jnp.maximum(ag[:, 5:6], 1.0)
    aggm = ag[:, 0:8] / cnt
    x = x_ref[...]
    z = (jnp.dot(aggm, wa_ref[...], preferred_element_type=jnp.float32)
         + jnp.dot(x, wx_ref[...], preferred_element_type=jnp.float32)
         + bcat_ref[...])
    mat_s = z[:, :CLUS]
    mat_y = jnp.maximum(z[:, CLUS:], 0.0)
    m = jnp.max(mat_s, axis=1, keepdims=True)
    e = jnp.exp(mat_s - m)
    s = jnp.sum(e, axis=1, keepdims=True)
    S = e / s
    deg = deg_ref[0][:, 0:1] + deg_ref[1][:, 0:1]
    dinv = jnp.where(deg > 0, lax.rsqrt(jnp.maximum(deg, 1e-12)), 0.0)
    q_ref[...] = dinv * S
    s2_acc[0] += jnp.sum(S * S)
    out_acc[...] += lax.dot_general(S, mat_y, (((0,), (0,)), ((), ())),
                                    preferred_element_type=jnp.float32)

    @pl.when(i == _NB - 1)
    def _head():
        out = out_acc[...]
        agg2 = jnp.dot(jnp.sum(out, axis=0, keepdims=True) / CLUS,
                       w2e_w_ref[...], preferred_element_type=jnp.float32)
        mat_y2 = jnp.maximum(
            agg2 + jnp.dot(out, w2e_root_ref[...],
                           preferred_element_type=jnp.float32)
            + w2e_b_ref[...], 0.0)
        out2 = jnp.sum(mat_y2, axis=0, keepdims=True)
        h = jnp.maximum(
            jnp.dot(out2, lin1_w_ref[...], preferred_element_type=jnp.float32)
            + lin1_b_ref[...], 0.0)
        h2 = (jnp.dot(h, lin2_w_ref[...], preferred_element_type=jnp.float32)
              + lin2_b_ref[...])
        mh = jnp.max(h2, axis=1, keepdims=True)
        lse = mh + jnp.log(jnp.sum(jnp.exp(h2 - mh), axis=1, keepdims=True))
        logp_ref[...] = h2 - lse
        sums2_ref[0, 0] = s2_acc[0]


def _tc_main(agg, deg, x, wa, wx, bcat, w2e_w, w2e_root, w2e_b,
             lin1_w, lin1_b, lin2_w, lin2_b):
    full = lambda *shape: pl.BlockSpec(shape, lambda i: (0,) * len(shape))
    return pl.pallas_call(
        _tc_body,
        grid=(_NB,),
        in_specs=[
            pl.BlockSpec((NC, _RB, 16), lambda i: (0, i, 0)),
            pl.BlockSpec((NC, _RB, 16), lambda i: (0, i, 0)),
            pl.BlockSpec((_RB, 5), lambda i: (i, 0)),
            full(8, 192), full(5, 192), full(1, 192),
            full(FOU1, FOU1), full(FOU1, FOU1), full(1, FOU1),
            full(FOU1, 256), full(1, 256), full(256, 8), full(1, 8),
        ],
        out_specs=[
            pl.BlockSpec((_RB, CLUS), lambda i: (i, 0)),
            pl.BlockSpec(memory_space=pltpu.SMEM),
            pl.BlockSpec((1, 8), lambda i: (0, 0)),
        ],
        out_shape=[
            jax.ShapeDtypeStruct((N_NODES, CLUS), jnp.float32),
            jax.ShapeDtypeStruct((1, 1), jnp.float32),
            jax.ShapeDtypeStruct((1, 8), jnp.float32),
        ],
        scratch_shapes=[
            pltpu.VMEM((CLUS, FOU1), jnp.float32),
            pltpu.SMEM((1,), jnp.float32),
        ],
    )(agg, deg, x, wa, wx, bcat, w2e_w, w2e_root, w2e_b,
      lin1_w, lin1_b, lin2_w, lin2_b)


def kernel(x, edge_index, edge_wht,
           w1p_w, w1p_root, w1p_b, w1e_w, w1e_root, w1e_b,
           w2p_w, w2p_root, w2p_b, w2e_w, w2e_root, w2e_b,
           lin1_w, lin1_b, lin2_w, lin2_b):
    f32 = jnp.float32
    src = edge_index[0].astype(jnp.int32)
    dst = edge_index[1].astype(jnp.int32)
    w = edge_wht.reshape(-1).astype(f32)

    src_r = src.reshape(NW, NCH, CH)
    dst_r = dst.reshape(NW, NCH, CH)
    w_r = w.reshape(NW, EW)
    xflat = x.reshape(-1).astype(f32)
    z8 = jnp.zeros((ROWS_PER_TILE, 16), f32)
    z4 = jnp.zeros((ROWS_PER_TILE, 16), f32)
    stag0 = jnp.zeros((CH, 16), f32).at[:, 5].set(1.0)

    wa = jnp.zeros((8, 192), f32).at[0:5, 0:128].set(w1p_w).at[0:5, 128:].set(w1e_w)
    wx = jnp.concatenate([w1p_root, w1e_root], axis=1)
    bcat = jnp.concatenate([w1p_b, w1e_b]).reshape(1, 192)

    sc_pass1, sc_pass2 = _sc_kernels()
    agg, deg = sc_pass1(xflat, src_r, dst_r, w_r, z8, z4, stag0)
    q, sums2, logp = _tc_main(
        agg, deg, x, wa, wx, bcat,
        w2e_w, w2e_root, w2e_b.reshape(1, FOU1),
        lin1_w, lin1_b.reshape(1, 256), lin2_w, lin2_b.reshape(1, 8))
    parts = sc_pass2(q, src_r, dst_r, w_r)
    reg1 = (sums2[0, 0] - jnp.sum(parts)) / f32(N_NODES)
    return logp, reg1


# speedup vs baseline: 1.1931x; 1.1931x over previous
import functools

import jax
import jax.numpy as jnp
from jax import lax
from jax.experimental import pallas as pl
from jax.experimental.pallas import tpu as pltpu
from jax.experimental.pallas import tpu_sc as plsc

N_NODES = 10000
N_EDGES = 320000
CLUS = 128
FOU1 = 64

NC = 2
NS = 16
NW = NC * NS
EW = N_EDGES // NW
CH = 80
NCH = EW // CH
NZT = 10
ROWS_PER_TILE = N_NODES // NZT

def _sc_pass1_body(xflat_hbm, src_hbm, dst_hbm, w_hbm, z8_hbm, z4_hbm,
                   stag0_hbm,
                   agg_out, deg_out,
                   src_v, dst_v, wvec_v, wbuf0_v, wbuf1_v, x_v,
                   stag0_v, stag1_v, zb8_v, zb4_v,
                   agg_sh, deg_sh, sem_a0, sem_d0, sem_a1, sem_d1):
    cid = lax.axis_index("c")
    sid = lax.axis_index("s")
    wid = cid * NS + sid

    @pl.when(sid < NZT)
    def _zero():
        zrows = pl.ds(sid * ROWS_PER_TILE, ROWS_PER_TILE)
        pltpu.sync_copy(z8_hbm, zb8_v)
        pltpu.sync_copy(zb8_v, agg_sh.at[zrows])
        pltpu.sync_copy(z4_hbm, zb4_v)
        pltpu.sync_copy(zb4_v, deg_sh.at[zrows])
    pltpu.sync_copy(z4_hbm.at[pl.ds(0, CH)], wbuf0_v)
    pltpu.sync_copy(z4_hbm.at[pl.ds(0, CH)], wbuf1_v)

    pltpu.sync_copy(xflat_hbm, x_v)
    pltpu.sync_copy(src_hbm.at[wid], src_v)
    pltpu.sync_copy(dst_hbm.at[wid], dst_v)
    pltpu.sync_copy(stag0_hbm, stag0_v)
    pltpu.sync_copy(stag0_hbm, stag1_v)
    plsc.subcore_barrier()

    lanes = lax.iota(jnp.int32, 16)
    zeros16 = jnp.zeros((16,), jnp.int32)
    bufs = ((stag0_v, wbuf0_v, sem_a0, sem_d0), (stag1_v, wbuf1_v, sem_a1, sem_d1))

    def fill(k, b):
        stag, wbuf, _, _ = bufs[b]
        pltpu.sync_copy(w_hbm.at[wid, pl.ds(k * CH, CH)], wvec_v)

        def group(g, carry):
            s16 = src_v[k, pl.ds(g * 16, 16)]
            rows = g * 16 + lanes
            for c in range(5):
                xc = plsc.load_gather(x_v, [s16 * 5 + c])
                plsc.store_scatter(stag, [rows, jnp.full((16,), c, jnp.int32)], xc)
            wv = wvec_v[pl.ds(g * 16, 16)]
            plsc.store_scatter(wbuf, [rows, zeros16], wv)
            return carry

        lax.fori_loop(0, CH // 16, group, 0, unroll=False)

    def start(k, b):
        stag, wbuf, sa, sd = bufs[b]
        pltpu.async_copy(stag, agg_sh.at[dst_v.at[k]], sa, add=True)
        pltpu.async_copy(wbuf, deg_sh.at[src_v.at[k]], sd, add=True)

    def wait(k, b):
        stag, wbuf, sa, sd = bufs[b]
        pltpu.make_async_copy(stag, agg_sh.at[dst_v.at[k]], sa).wait()
        pltpu.make_async_copy(wbuf, deg_sh.at[src_v.at[k]], sd).wait()

    fill(0, 0)
    start(0, 0)

    def pair(j, carry):
        k0 = 2 * j
        fill(k0 + 1, 1)
        start(k0 + 1, 1)
        wait(k0, 0)

        @pl.when(j + 1 < NCH // 2)
        def _next():
            fill(k0 + 2, 0)
            start(k0 + 2, 0)

        wait(k0 + 1, 1)
        return carry

    lax.fori_loop(0, NCH // 2, pair, 0, unroll=False)
    if NCH % 2:
        fill(NCH - 1, 0)
        start(NCH - 1, 0)
        wait(NCH - 1, 0)
    plsc.subcore_barrier()

    @pl.when(sid < NZT)
    def _writeout():
        rows = pl.ds(sid * ROWS_PER_TILE, ROWS_PER_TILE)
        pltpu.sync_copy(agg_sh.at[rows], zb8_v)
        pltpu.sync_copy(zb8_v, agg_out.at[cid, rows])
        pltpu.sync_copy(deg_sh.at[rows], zb4_v)
        pltpu.sync_copy(zb4_v, deg_out.at[cid, rows])


def _sc_pass2_body(q_hbm, src_hbm, dst_hbm, w_hbm, part_out,
                   src_v, dst_v, w_v,
                   qa0_v, qb0_v, qa1_v, qb1_v, acc_v,
                   sem_a0, sem_b0, sem_a1, sem_b1):
    cid = lax.axis_index("c")
    sid = lax.axis_index("s")
    wid = cid * NS + sid

    pltpu.sync_copy(src_hbm.at[wid], src_v)
    pltpu.sync_copy(dst_hbm.at[wid], dst_v)
    pltpu.sync_copy(w_hbm.at[wid], w_v)

    bufs = ((qa0_v, qb0_v, sem_a0, sem_b0), (qa1_v, qb1_v, sem_a1, sem_b1))

    def start(k, b):
        qa, qb, sa, sb = bufs[b]
        pltpu.async_copy(q_hbm.at[src_v.at[k]], qa, sa)
        pltpu.async_copy(q_hbm.at[dst_v.at[k]], qb, sb)

    def compute(k, b, acc):
        qa, qb, sa, sb = bufs[b]
        pltpu.make_async_copy(q_hbm.at[src_v.at[k]], qa, sa).wait()
        pltpu.make_async_copy(q_hbm.at[dst_v.at[k]], qb, sb).wait()

        def row(r, acc):
            wbc = plsc.load_gather(w_v, [jnp.full((16,), k * CH + r, jnp.int32)])
            d = qa[r, pl.ds(0, 16)] * qb[r, pl.ds(0, 16)]
            for u in range(1, CLUS // 16):
                d = d + qa[r, pl.ds(u * 16, 16)] * qb[r, pl.ds(u * 16, 16)]
            return acc + wbc * d

        return lax.fori_loop(0, CH, row, acc, unroll=8)

    start(0, 0)

    def pair(j, acc):
        k0 = 2 * j
        start(k0 + 1, 1)
        acc = compute(k0, 0, acc)

        @pl.when(j + 1 < NCH // 2)
        def _next():
            start(k0 + 2, 0)

        return compute(k0 + 1, 1, acc)

    acc = lax.fori_loop(0, NCH // 2, pair, jnp.zeros((16,), jnp.float32),
                        unroll=False)
    if NCH % 2:
        start(NCH - 1, 0)
        acc = compute(NCH - 1, 0, acc)
    acc_v[...] = acc
    pltpu.sync_copy(acc_v, part_out.at[pl.ds(wid * 16, 16)])


@functools.cache
def _sc_kernels():
    mesh = plsc.VectorSubcoreMesh(core_axis_name="c", subcore_axis_name="s")
    params = pltpu.CompilerParams(needs_layout_passes=False, use_tc_tiling_on_sc=False)
    pass1 = pl.kernel(
        _sc_pass1_body,
        mesh=mesh,
        compiler_params=params,
        out_type=[
            jax.ShapeDtypeStruct((NC, N_NODES, 16), jnp.float32),
            jax.ShapeDtypeStruct((NC, N_NODES, 16), jnp.float32),
        ],
        scratch_types=[
            pltpu.VMEM((NCH, CH), jnp.int32),
            pltpu.VMEM((NCH, CH), jnp.int32),
            pltpu.VMEM((CH,), jnp.float32),
            pltpu.VMEM((CH, 16), jnp.float32),
            pltpu.VMEM((CH, 16), jnp.float32),
            pltpu.VMEM((N_NODES * 5,), jnp.float32),
            pltpu.VMEM((CH, 16), jnp.float32),
            pltpu.VMEM((CH, 16), jnp.float32),
            pltpu.VMEM((ROWS_PER_TILE, 16), jnp.float32),
            pltpu.VMEM((ROWS_PER_TILE, 16), jnp.float32),
            pltpu.VMEM_SHARED((N_NODES, 16), jnp.float32),
            pltpu.VMEM_SHARED((N_NODES, 16), jnp.float32),
            pltpu.SemaphoreType.DMA,
            pltpu.SemaphoreType.DMA,
            pltpu.SemaphoreType.DMA,
            pltpu.SemaphoreType.DMA,
        ],
    )
    pass2 = pl.kernel(
        _sc_pass2_body,
        mesh=mesh,
        compiler_params=params,
        out_type=jax.ShapeDtypeStruct((NW * 16,), jnp.float32),
        scratch_types=[
            pltpu.VMEM((NCH, CH), jnp.int32),
            pltpu.VMEM((NCH, CH), jnp.int32),
            pltpu.VMEM((EW,), jnp.float32),
            pltpu.VMEM((CH, CLUS), jnp.float32),
            pltpu.VMEM((CH, CLUS), jnp.float32),
            pltpu.VMEM((CH, CLUS), jnp.float32),
            pltpu.VMEM((CH, CLUS), jnp.float32),
            pltpu.VMEM((16,), jnp.float32),
            pltpu.SemaphoreType.DMA,
            pltpu.SemaphoreType.DMA,
            pltpu.SemaphoreType.DMA,
            pltpu.SemaphoreType.DMA,
        ],
    )
    return pass1, pass2


_RB = 2000
_NB = N_NODES // _RB


def _tc_body(agg_ref, deg_ref, x_ref, wa_ref, wx_ref, bcat_ref,
             w2e_w_ref, w2e_root_ref, w2e_b_ref,
             lin1_w_ref, lin1_b_ref, lin2_w_ref, lin2_b_ref,
             q_ref, sums2_ref, logp_ref,
             out_acc, s2_acc):
    i = pl.program_id(0)

    @pl.when(i == 0)
    def _init():
        out_acc[...] = jnp.zeros_like(out_acc)
        s2_acc[0] = 0.0

    ag = agg_ref[0] + agg_ref[1]
    cnt = jnp.maximum(ag[:, 5:6], 1.0)
    aggm = ag[:, 0:8] / cnt
    x = x_ref[...]
    z = (jnp.dot(aggm, wa_ref[...], preferred_element_type=jnp.float32)
         + jnp.dot(x, wx_ref[...], preferred_element_type=jnp.float32)
         + bcat_ref[...])
    mat_s = z[:, :CLUS]
    mat_y = jnp.maximum(z[:, CLUS:], 0.0)
    m = jnp.max(mat_s, axis=1, keepdims=True)
    e = jnp.exp(mat_s - m)
    s = jnp.sum(e, axis=1, keepdims=True)
    S = e / s
    deg = deg_ref[0][:, 0:1] + deg_ref[1][:, 0:1]
    dinv = jnp.where(deg > 0, lax.rsqrt(jnp.maximum(deg, 1e-12)), 0.0)
    q_ref[...] = dinv * S
    s2_acc[0] += jnp.sum(S * S)
    out_acc[...] += lax.dot_general(S, mat_y, (((0,), (0,)), ((), ())),
                                    preferred_element_type=jnp.float32)

    @pl.when(i == _NB - 1)
    def _head():
        out = out_acc[...]
        agg2 = jnp.dot(jnp.sum(out, axis=0, keepdims=True) / CLUS,
                       w2e_w_ref[...], preferred_element_type=jnp.float32)
        mat_y2 = jnp.maximum(
            agg2 + jnp.dot(out, w2e_root_ref[...],
                           preferred_element_type=jnp.float32)
            + w2e_b_ref[...], 0.0)
        out2 = jnp.sum(mat_y2, axis=0, keepdims=True)
        h = jnp.maximum(
            jnp.dot(out2, lin1_w_ref[...], preferred_element_type=jnp.float32)
            + lin1_b_ref[...], 0.0)
        h2 = (jnp.dot(h, lin2_w_ref[...], preferred_element_type=jnp.float32)
              + lin2_b_ref[...])
        mh = jnp.max(h2, axis=1, keepdims=True)
        lse = mh + jnp.log(jnp.sum(jnp.exp(h2 - mh), axis=1, keepdims=True))
        logp_ref[...] = h2 - lse
        sums2_ref[0, 0] = s2_acc[0]


def _tc_main(agg, deg, x, wa, wx, bcat, w2e_w, w2e_root, w2e_b,
             lin1_w, lin1_b, lin2_w, lin2_b):
    full = lambda *shape: pl.BlockSpec(shape, lambda i: (0,) * len(shape))
    return pl.pallas_call(
        _tc_body,
        grid=(_NB,),
        in_specs=[
            pl.BlockSpec((NC, _RB, 16), lambda i: (0, i, 0)),
            pl.BlockSpec((NC, _RB, 16), lambda i: (0, i, 0)),
            pl.BlockSpec((_RB, 5), lambda i: (i, 0)),
            full(8, 192), full(5, 192), full(1, 192),
            full(FOU1, FOU1), full(FOU1, FOU1), full(1, FOU1),
            full(FOU1, 256), full(1, 256), full(256, 8), full(1, 8),
        ],
        out_specs=[
            pl.BlockSpec((_RB, CLUS), lambda i: (i, 0)),
            pl.BlockSpec(memory_space=pltpu.SMEM),
            pl.BlockSpec((1, 8), lambda i: (0, 0)),
        ],
        out_shape=[
            jax.ShapeDtypeStruct((N_NODES, CLUS), jnp.float32),
            jax.ShapeDtypeStruct((1, 1), jnp.float32),
            jax.ShapeDtypeStruct((1, 8), jnp.float32),
        ],
        scratch_shapes=[
            pltpu.VMEM((CLUS, FOU1), jnp.float32),
            pltpu.SMEM((1,), jnp.float32),
        ],
    )(agg, deg, x, wa, wx, bcat, w2e_w, w2e_root, w2e_b,
      lin1_w, lin1_b, lin2_w, lin2_b)


def kernel(x, edge_index, edge_wht,
           w1p_w, w1p_root, w1p_b, w1e_w, w1e_root, w1e_b,
           w2p_w, w2p_root, w2p_b, w2e_w, w2e_root, w2e_b,
           lin1_w, lin1_b, lin2_w, lin2_b):
    f32 = jnp.float32
    src = edge_index[0].astype(jnp.int32)
    dst = edge_index[1].astype(jnp.int32)
    w = edge_wht.reshape(-1).astype(f32)

    src_r = src.reshape(NW, NCH, CH)
    dst_r = dst.reshape(NW, NCH, CH)
    w_r = w.reshape(NW, EW)
    xflat = x.reshape(-1).astype(f32)
    z8 = jnp.zeros((ROWS_PER_TILE, 16), f32)
    z4 = jnp.zeros((ROWS_PER_TILE, 16), f32)
    stag0 = jnp.zeros((CH, 16), f32).at[:, 5].set(1.0)

    wa = jnp.zeros((8, 192), f32).at[0:5, 0:128].set(w1p_w).at[0:5, 128:].set(w1e_w)
    wx = jnp.concatenate([w1p_root, w1e_root], axis=1)
    bcat = jnp.concatenate([w1p_b, w1e_b]).reshape(1, 192)

    sc_pass1, sc_pass2 = _sc_kernels()
    agg, deg = sc_pass1(xflat, src_r, dst_r, w_r, z8, z4, stag0)
    q, sums2, logp = _tc_main(
        agg, deg, x, wa, wx, bcat,
        w2e_w, w2e_root, w2e_b.reshape(1, FOU1),
        lin1_w, lin1_b.reshape(1, 256), lin2_w, lin2_b.reshape(1, 8))
    parts = sc_pass2(q, src_r, dst_r, w_r)
    reg1 = (sums2[0, 0] - jnp.sum(parts)) / f32(N_NODES)
    return logp, reg1


# speedup vs baseline: 1.4261x; 1.1954x over previous
import functools

import jax
import jax.numpy as jnp
from jax import lax
from jax.experimental import pallas as pl
from jax.experimental.pallas import tpu as pltpu
from jax.experimental.pallas import tpu_sc as plsc

N_NODES = 10000
N_EDGES = 320000
CLUS = 128
FOU1 = 64

NC = 2
NS = 16
NW = NC * NS
EW = N_EDGES // NW
CH = 80
NCH = EW // CH
NZT = 10
ROWS_PER_TILE = N_NODES // NZT

def _sc_pass1_body(xflat_hbm, src_hbm, dst_hbm, w_hbm, z8_hbm, z4_hbm,
                   stag0_hbm,
                   agg_out, deg_out,
                   src_v, dst_v, wvec_v, wbuf0_v, wbuf1_v, x_v,
                   stag0_v, stag1_v, zb8_v, zb4_v,
                   agg_sh, deg_sh, sem_a0, sem_d0, sem_a1, sem_d1):
    cid = lax.axis_index("c")
    sid = lax.axis_index("s")
    wid = cid * NS + sid

    @pl.when(sid < NZT)
    def _zero():
        zrows = pl.ds(sid * ROWS_PER_TILE, ROWS_PER_TILE)
        pltpu.sync_copy(z8_hbm, zb8_v)
        pltpu.sync_copy(zb8_v, agg_sh.at[zrows])
        pltpu.sync_copy(z4_hbm, zb4_v)
        pltpu.sync_copy(zb4_v, deg_sh.at[zrows])
    pltpu.sync_copy(z4_hbm.at[pl.ds(0, CH)], wbuf0_v)
    pltpu.sync_copy(z4_hbm.at[pl.ds(0, CH)], wbuf1_v)

    pltpu.sync_copy(xflat_hbm, x_v)
    pltpu.sync_copy(src_hbm.at[wid], src_v)
    pltpu.sync_copy(dst_hbm.at[wid], dst_v)
    pltpu.sync_copy(stag0_hbm, stag0_v)
    pltpu.sync_copy(stag0_hbm, stag1_v)
    plsc.subcore_barrier()

    lanes = lax.iota(jnp.int32, 16)
    zeros16 = jnp.zeros((16,), jnp.int32)
    bufs = ((stag0_v, wbuf0_v, sem_a0, sem_d0), (stag1_v, wbuf1_v, sem_a1, sem_d1))

    def fill(k, b):
        stag, wbuf, _, _ = bufs[b]
        pltpu.sync_copy(w_hbm.at[wid, pl.ds(k * CH, CH)], wvec_v)

        def group(g, carry):
            s16 = src_v[k, pl.ds(g * 16, 16)]
            rows = g * 16 + lanes
            for c in range(5):
                xc = plsc.load_gather(x_v, [s16 * 5 + c])
                plsc.store_scatter(stag, [rows, jnp.full((16,), c, jnp.int32)], xc)
            wv = wvec_v[pl.ds(g * 16, 16)]
            plsc.store_scatter(wbuf, [rows, zeros16], wv)
            return carry

        lax.fori_loop(0, CH // 16, group, 0, unroll=False)

    def start(k, b):
        stag, wbuf, sa, sd = bufs[b]
        pltpu.async_copy(stag, agg_sh.at[dst_v.at[k]], sa, add=True)
        pltpu.async_copy(wbuf, deg_sh.at[src_v.at[k]], sd, add=True)

    def wait(k, b):
        stag, wbuf, sa, sd = bufs[b]
        pltpu.make_async_copy(stag, agg_sh.at[dst_v.at[k]], sa).wait()
        pltpu.make_async_copy(wbuf, deg_sh.at[src_v.at[k]], sd).wait()

    fill(0, 0)
    start(0, 0)

    def pair(j, carry):
        k0 = 2 * j
        fill(k0 + 1, 1)
        start(k0 + 1, 1)
        wait(k0, 0)

        @pl.when(j + 1 < NCH // 2)
        def _next():
            fill(k0 + 2, 0)
            start(k0 + 2, 0)

        wait(k0 + 1, 1)
        return carry

    lax.fori_loop(0, NCH // 2, pair, 0, unroll=False)
    if NCH % 2:
        fill(NCH - 1, 0)
        start(NCH - 1, 0)
        wait(NCH - 1, 0)
    plsc.subcore_barrier()

    @pl.when(sid < NZT)
    def _writeout():
        rows = pl.ds(sid * ROWS_PER_TILE, ROWS_PER_TILE)
        pltpu.sync_copy(agg_sh.at[rows], zb8_v)
        pltpu.sync_copy(zb8_v, agg_out.at[cid, rows])
        pltpu.sync_copy(deg_sh.at[rows], zb4_v)
        pltpu.sync_copy(zb4_v, deg_out.at[cid, rows])


def _sc_pass2_body(q_hbm, src_hbm, dst_hbm, w_hbm, part_out,
                   src_v, dst_v, w_v,
                   qa0_v, qb0_v, qa1_v, qb1_v, acc_v,
                   sem_a0, sem_b0, sem_a1, sem_b1):
    cid = lax.axis_index("c")
    sid = lax.axis_index("s")
    wid = cid * NS + sid

    pltpu.sync_copy(src_hbm.at[wid], src_v)
    pltpu.sync_copy(dst_hbm.at[wid], dst_v)
    pltpu.sync_copy(w_hbm.at[wid], w_v)

    bufs = ((qa0_v, qb0_v, sem_a0, sem_b0), (qa1_v, qb1_v, sem_a1, sem_b1))

    def start(k, b):
        qa, qb, sa, sb = bufs[b]
        pltpu.async_copy(q_hbm.at[src_v.at[k]], qa, sa)
        pltpu.async_copy(q_hbm.at[dst_v.at[k]], qb, sb)

    def compute(k, b, acc):
        qa, qb, sa, sb = bufs[b]
        pltpu.make_async_copy(q_hbm.at[src_v.at[k]], qa, sa).wait()
        pltpu.make_async_copy(q_hbm.at[dst_v.at[k]], qb, sb).wait()

        def row(r, acc):
            wbc = plsc.load_gather(w_v, [jnp.full((16,), k * CH + r, jnp.int32)])
            d = qa[r, pl.ds(0, 16)] * qb[r, pl.ds(0, 16)]
            for u in range(1, CLUS // 16):
                d = d + qa[r, pl.ds(u * 16, 16)] * qb[r, pl.ds(u * 16, 16)]
            return acc + wbc * d

        return lax.fori_loop(0, CH, row, acc, unroll=4)

    start(0, 0)

    def pair(j, acc):
        k0 = 2 * j
        start(k0 + 1, 1)
        acc = compute(k0, 0, acc)

        @pl.when(j + 1 < NCH // 2)
        def _next():
            start(k0 + 2, 0)

        return compute(k0 + 1, 1, acc)

    acc = lax.fori_loop(0, NCH // 2, pair, jnp.zeros((16,), jnp.float32),
                        unroll=False)
    if NCH % 2:
        start(NCH - 1, 0)
        acc = compute(NCH - 1, 0, acc)
    acc_v[...] = acc
    pltpu.sync_copy(acc_v, part_out.at[pl.ds(wid * 16, 16)])


@functools.cache
def _sc_kernels():
    mesh = plsc.VectorSubcoreMesh(core_axis_name="c", subcore_axis_name="s")
    params = pltpu.CompilerParams(needs_layout_passes=False, use_tc_tiling_on_sc=False)
    pass1 = pl.kernel(
        _sc_pass1_body,
        mesh=mesh,
        compiler_params=params,
        out_type=[
            jax.ShapeDtypeStruct((NC, N_NODES, 16), jnp.float32),
            jax.ShapeDtypeStruct((NC, N_NODES, 16), jnp.float32),
        ],
        scratch_types=[
            pltpu.VMEM((NCH, CH), jnp.int32),
            pltpu.VMEM((NCH, CH), jnp.int32),
            pltpu.VMEM((CH,), jnp.float32),
            pltpu.VMEM((CH, 16), jnp.float32),
            pltpu.VMEM((CH, 16), jnp.float32),
            pltpu.VMEM((N_NODES * 5,), jnp.float32),
            pltpu.VMEM((CH, 16), jnp.float32),
            pltpu.VMEM((CH, 16), jnp.float32),
            pltpu.VMEM((ROWS_PER_TILE, 16), jnp.float32),
            pltpu.VMEM((ROWS_PER_TILE, 16), jnp.float32),
            pltpu.VMEM_SHARED((N_NODES, 16), jnp.float32),
            pltpu.VMEM_SHARED((N_NODES, 16), jnp.float32),
            pltpu.SemaphoreType.DMA,
            pltpu.SemaphoreType.DMA,
            pltpu.SemaphoreType.DMA,
            pltpu.SemaphoreType.DMA,
        ],
    )
    pass2 = pl.kernel(
        _sc_pass2_body,
        mesh=mesh,
        compiler_params=params,
        out_type=jax.ShapeDtypeStruct((NW * 16,), jnp.float32),
        scratch_types=[
            pltpu.VMEM((NCH, CH), jnp.int32),
            pltpu.VMEM((NCH, CH), jnp.int32),
            pltpu.VMEM((EW,), jnp.float32),
            pltpu.VMEM((CH, CLUS), jnp.float32),
            pltpu.VMEM((CH, CLUS), jnp.float32),
            pltpu.VMEM((CH, CLUS), jnp.float32),
            pltpu.VMEM((CH, CLUS), jnp.float32),
            pltpu.VMEM((16,), jnp.float32),
            pltpu.SemaphoreType.DMA,
            pltpu.SemaphoreType.DMA,
            pltpu.SemaphoreType.DMA,
            pltpu.SemaphoreType.DMA,
        ],
    )
    return pass1, pass2


_RB = 2000
_NB = N_NODES // _RB


def _tc_body(agg_ref, deg_ref, x_ref, wa_ref, wx_ref, bcat_ref,
             w2e_w_ref, w2e_root_ref, w2e_b_ref,
             lin1_w_ref, lin1_b_ref, lin2_w_ref, lin2_b_ref,
             q_ref, sums2_ref, logp_ref,
             out_acc, s2_acc):
    i = pl.program_id(0)

    @pl.when(i == 0)
    def _init():
        out_acc[...] = jnp.zeros_like(out_acc)
        s2_acc[0] = 0.0

    ag = agg_ref[0] + agg_ref[1]
    cnt = jnp.maximum(ag[:, 5:6], 1.0)
    aggm = ag[:, 0:8] / cnt
    x = x_ref[...]
    z = (jnp.dot(aggm, wa_ref[...], preferred_element_type=jnp.float32)
         + jnp.dot(x, wx_ref[...], preferred_element_type=jnp.float32)
         + bcat_ref[...])
    mat_s = z[:, :CLUS]
    mat_y = jnp.maximum(z[:, CLUS:], 0.0)
    m = jnp.max(mat_s, axis=1, keepdims=True)
    e = jnp.exp(mat_s - m)
    s = jnp.sum(e, axis=1, keepdims=True)
    S = e / s
    deg = deg_ref[0][:, 0:1] + deg_ref[1][:, 0:1]
    dinv = jnp.where(deg > 0, lax.rsqrt(jnp.maximum(deg, 1e-12)), 0.0)
    q_ref[...] = dinv * S
    s2_acc[0] += jnp.sum(S * S)
    out_acc[...] += lax.dot_general(S, mat_y, (((0,), (0,)), ((), ())),
                                    preferred_element_type=jnp.float32)

    @pl.when(i == _NB - 1)
    def _head():
        out = out_acc[...]
        agg2 = jnp.dot(jnp.sum(out, axis=0, keepdims=True) / CLUS,
                       w2e_w_ref[...], preferred_element_type=jnp.float32)
        mat_y2 = jnp.maximum(
            agg2 + jnp.dot(out, w2e_root_ref[...],
                           preferred_element_type=jnp.float32)
            + w2e_b_ref[...], 0.0)
        out2 = jnp.sum(mat_y2, axis=0, keepdims=True)
        h = jnp.maximum(
            jnp.dot(out2, lin1_w_ref[...], preferred_element_type=jnp.float32)
            + lin1_b_ref[...], 0.0)
        h2 = (jnp.dot(h, lin2_w_ref[...], preferred_element_type=jnp.float32)
              + lin2_b_ref[...])
        mh = jnp.max(h2, axis=1, keepdims=True)
        lse = mh + jnp.log(jnp.sum(jnp.exp(h2 - mh), axis=1, keepdims=True))
        logp_ref[...] = h2 - lse
        sums2_ref[0, 0] = s2_acc[0]


def _tc_main(agg, deg, x, wa, wx, bcat, w2e_w, w2e_root, w2e_b,
             lin1_w, lin1_b, lin2_w, lin2_b):
    full = lambda *shape: pl.BlockSpec(shape, lambda i: (0,) * len(shape))
    return pl.pallas_call(
        _tc_body,
        grid=(_NB,),
        in_specs=[
            pl.BlockSpec((NC, _RB, 16), lambda i: (0, i, 0)),
            pl.BlockSpec((NC, _RB, 16), lambda i: (0, i, 0)),
            pl.BlockSpec((_RB, 5), lambda i: (i, 0)),
            full(8, 192), full(5, 192), full(1, 192),
            full(FOU1, FOU1), full(FOU1, FOU1), full(1, FOU1),
            full(FOU1, 256), full(1, 256), full(256, 8), full(1, 8),
        ],
        out_specs=[
            pl.BlockSpec((_RB, CLUS), lambda i: (i, 0)),
            pl.BlockSpec(memory_space=pltpu.SMEM),
            pl.BlockSpec((1, 8), lambda i: (0, 0)),
        ],
        out_shape=[
            jax.ShapeDtypeStruct((N_NODES, CLUS), jnp.float32),
            jax.ShapeDtypeStruct((1, 1), jnp.float32),
            jax.ShapeDtypeStruct((1, 8), jnp.float32),
        ],
        scratch_shapes=[
            pltpu.VMEM((CLUS, FOU1), jnp.float32),
            pltpu.SMEM((1,), jnp.float32),
        ],
    )(agg, deg, x, wa, wx, bcat, w2e_w, w2e_root, w2e_b,
      lin1_w, lin1_b, lin2_w, lin2_b)


def kernel(x, edge_index, edge_wht,
           w1p_w, w1p_root, w1p_b, w1e_w, w1e_root, w1e_b,
           w2p_w, w2p_root, w2p_b, w2e_w, w2e_root, w2e_b,
           lin1_w, lin1_b, lin2_w, lin2_b):
    f32 = jnp.float32
    src = edge_index[0].astype(jnp.int32)
    dst = edge_index[1].astype(jnp.int32)
    w = edge_wht.reshape(-1).astype(f32)

    src_r = src.reshape(NW, NCH, CH)
    dst_r = dst.reshape(NW, NCH, CH)
    w_r = w.reshape(NW, EW)
    xflat = x.reshape(-1).astype(f32)
    z8 = jnp.zeros((ROWS_PER_TILE, 16), f32)
    z4 = jnp.zeros((ROWS_PER_TILE, 16), f32)
    stag0 = jnp.zeros((CH, 16), f32).at[:, 5].set(1.0)

    wa = jnp.zeros((8, 192), f32).at[0:5, 0:128].set(w1p_w).at[0:5, 128:].set(w1e_w)
    wx = jnp.concatenate([w1p_root, w1e_root], axis=1)
    bcat = jnp.concatenate([w1p_b, w1e_b]).reshape(1, 192)

    sc_pass1, sc_pass2 = _sc_kernels()
    agg, deg = sc_pass1(xflat, src_r, dst_r, w_r, z8, z4, stag0)
    q, sums2, logp = _tc_main(
        agg, deg, x, wa, wx, bcat,
        w2e_w, w2e_root, w2e_b.reshape(1, FOU1),
        lin1_w, lin1_b.reshape(1, 256), lin2_w, lin2_b.reshape(1, 8))
    parts = sc_pass2(q, src_r, dst_r, w_r)
    reg1 = (sums2[0, 0] - jnp.sum(parts)) / f32(N_NODES)
    return logp, reg1


# speedup vs baseline: 1.4284x; 1.0016x over previous
import functools

import jax
import jax.numpy as jnp
from jax import lax
from jax.experimental import pallas as pl
from jax.experimental.pallas import tpu as pltpu
from jax.experimental.pallas import tpu_sc as plsc

N_NODES = 10000
N_EDGES = 320000
CLUS = 128
FOU1 = 64

NC = 2
NS = 16
NW = NC * NS
EW = N_EDGES // NW
CH = 80
NCH = EW // CH
NZT = 10
ROWS_PER_TILE = N_NODES // NZT

def _sc_pass1_body(xflat_hbm, src_hbm, dst_hbm, w_hbm, z8_hbm, z4_hbm,
                   stag0_hbm,
                   agg_out, deg_out,
                   src_v, dst_v, wvec_v, wbuf0_v, wbuf1_v, x_v,
                   stag0_v, stag1_v, zb8_v, zb4_v,
                   agg_sh, deg_sh, sem_a0, sem_d0, sem_a1, sem_d1):
    cid = lax.axis_index("c")
    sid = lax.axis_index("s")
    wid = cid * NS + sid

    @pl.when(sid < NZT)
    def _zero():
        zrows = pl.ds(sid * ROWS_PER_TILE, ROWS_PER_TILE)
        pltpu.sync_copy(z8_hbm, zb8_v)
        pltpu.sync_copy(zb8_v, agg_sh.at[zrows])
        pltpu.sync_copy(z4_hbm, zb4_v)
        pltpu.sync_copy(zb4_v, deg_sh.at[zrows])
    pltpu.sync_copy(z4_hbm.at[pl.ds(0, CH)], wbuf0_v)
    pltpu.sync_copy(z4_hbm.at[pl.ds(0, CH)], wbuf1_v)

    pltpu.sync_copy(xflat_hbm, x_v)
    pltpu.sync_copy(src_hbm.at[wid], src_v)
    pltpu.sync_copy(dst_hbm.at[wid], dst_v)
    pltpu.sync_copy(stag0_hbm, stag0_v)
    pltpu.sync_copy(stag0_hbm, stag1_v)
    plsc.subcore_barrier()

    lanes = lax.iota(jnp.int32, 16)
    zeros16 = jnp.zeros((16,), jnp.int32)
    bufs = ((stag0_v, wbuf0_v, sem_a0, sem_d0), (stag1_v, wbuf1_v, sem_a1, sem_d1))

    def fill(k, b):
        stag, wbuf, _, _ = bufs[b]
        pltpu.sync_copy(w_hbm.at[wid, pl.ds(k * CH, CH)], wvec_v)

        def group(g, carry):
            s16 = src_v[k, pl.ds(g * 16, 16)]
            rows = g * 16 + lanes
            for c in range(5):
                xc = plsc.load_gather(x_v, [s16 * 5 + c])
                plsc.store_scatter(stag, [rows, jnp.full((16,), c, jnp.int32)], xc)
            wv = wvec_v[pl.ds(g * 16, 16)]
            plsc.store_scatter(wbuf, [rows, zeros16], wv)
            return carry

        lax.fori_loop(0, CH // 16, group, 0, unroll=False)

    def start(k, b):
        stag, wbuf, sa, sd = bufs[b]
        pltpu.async_copy(stag, agg_sh.at[dst_v.at[k]], sa, add=True)
        pltpu.async_copy(wbuf, deg_sh.at[src_v.at[k]], sd, add=True)

    def wait(k, b):
        stag, wbuf, sa, sd = bufs[b]
        pltpu.make_async_copy(stag, agg_sh.at[dst_v.at[k]], sa).wait()
        pltpu.make_async_copy(wbuf, deg_sh.at[src_v.at[k]], sd).wait()

    fill(0, 0)
    start(0, 0)

    def pair(j, carry):
        k0 = 2 * j
        fill(k0 + 1, 1)
        start(k0 + 1, 1)
        wait(k0, 0)

        @pl.when(j + 1 < NCH // 2)
        def _next():
            fill(k0 + 2, 0)
            start(k0 + 2, 0)

        wait(k0 + 1, 1)
        return carry

    lax.fori_loop(0, NCH // 2, pair, 0, unroll=False)
    if NCH % 2:
        fill(NCH - 1, 0)
        start(NCH - 1, 0)
        wait(NCH - 1, 0)
    plsc.subcore_barrier()

    @pl.when(sid < NZT)
    def _writeout():
        rows = pl.ds(sid * ROWS_PER_TILE, ROWS_PER_TILE)
        pltpu.sync_copy(agg_sh.at[rows], zb8_v)
        pltpu.sync_copy(zb8_v, agg_out.at[cid, rows])
        pltpu.sync_copy(deg_sh.at[rows], zb4_v)
        pltpu.sync_copy(zb4_v, deg_out.at[cid, rows])


def _sc_pass2_body(q_hbm, src_hbm, dst_hbm, w_hbm, part_out,
                   src_v, dst_v, w_v,
                   qa0_v, qb0_v, qa1_v, qb1_v, acc_v,
                   sem_a0, sem_b0, sem_a1, sem_b1):
    cid = lax.axis_index("c")
    sid = lax.axis_index("s")
    wid = cid * NS + sid

    pltpu.sync_copy(src_hbm.at[wid], src_v)
    pltpu.sync_copy(dst_hbm.at[wid], dst_v)
    pltpu.sync_copy(w_hbm.at[wid], w_v)

    bufs = ((qa0_v, qb0_v, sem_a0, sem_b0), (qa1_v, qb1_v, sem_a1, sem_b1))

    def start(k, b):
        qa, qb, sa, sb = bufs[b]
        pltpu.async_copy(q_hbm.at[src_v.at[k]], qa, sa)
        pltpu.async_copy(q_hbm.at[dst_v.at[k]], qb, sb)

    def compute(k, b, acc):
        qa, qb, sa, sb = bufs[b]
        pltpu.make_async_copy(q_hbm.at[src_v.at[k]], qa, sa).wait()
        pltpu.make_async_copy(q_hbm.at[dst_v.at[k]], qb, sb).wait()

        def row(r, acc):
            wbc = plsc.load_gather(w_v, [jnp.full((16,), k * CH + r, jnp.int32)])
            d = qa[r, pl.ds(0, 16)] * qb[r, pl.ds(0, 16)]
            for u in range(1, CLUS // 16):
                d = d + qa[r, pl.ds(u * 16, 16)] * qb[r, pl.ds(u * 16, 16)]
            return acc + wbc * d

        return lax.fori_loop(0, CH, row, acc, unroll=2)

    start(0, 0)

    def pair(j, acc):
        k0 = 2 * j
        start(k0 + 1, 1)
        acc = compute(k0, 0, acc)

        @pl.when(j + 1 < NCH // 2)
        def _next():
            start(k0 + 2, 0)

        return compute(k0 + 1, 1, acc)

    acc = lax.fori_loop(0, NCH // 2, pair, jnp.zeros((16,), jnp.float32),
                        unroll=False)
    if NCH % 2:
        start(NCH - 1, 0)
        acc = compute(NCH - 1, 0, acc)
    acc_v[...] = acc
    pltpu.sync_copy(acc_v, part_out.at[pl.ds(wid * 16, 16)])


@functools.cache
def _sc_kernels():
    mesh = plsc.VectorSubcoreMesh(core_axis_name="c", subcore_axis_name="s")
    params = pltpu.CompilerParams(needs_layout_passes=False, use_tc_tiling_on_sc=False)
    pass1 = pl.kernel(
        _sc_pass1_body,
        mesh=mesh,
        compiler_params=params,
        out_type=[
            jax.ShapeDtypeStruct((NC, N_NODES, 16), jnp.float32),
            jax.ShapeDtypeStruct((NC, N_NODES, 16), jnp.float32),
        ],
        scratch_types=[
            pltpu.VMEM((NCH, CH), jnp.int32),
            pltpu.VMEM((NCH, CH), jnp.int32),
            pltpu.VMEM((CH,), jnp.float32),
            pltpu.VMEM((CH, 16), jnp.float32),
            pltpu.VMEM((CH, 16), jnp.float32),
            pltpu.VMEM((N_NODES * 5,), jnp.float32),
            pltpu.VMEM((CH, 16), jnp.float32),
            pltpu.VMEM((CH, 16), jnp.float32),
            pltpu.VMEM((ROWS_PER_TILE, 16), jnp.float32),
            pltpu.VMEM((ROWS_PER_TILE, 16), jnp.float32),
            pltpu.VMEM_SHARED((N_NODES, 16), jnp.float32),
            pltpu.VMEM_SHARED((N_NODES, 16), jnp.float32),
            pltpu.SemaphoreType.DMA,
            pltpu.SemaphoreType.DMA,
            pltpu.SemaphoreType.DMA,
            pltpu.SemaphoreType.DMA,
        ],
    )
    pass2 = pl.kernel(
        _sc_pass2_body,
        mesh=mesh,
        compiler_params=params,
        out_type=jax.ShapeDtypeStruct((NW * 16,), jnp.float32),
        scratch_types=[
            pltpu.VMEM((NCH, CH), jnp.int32),
            pltpu.VMEM((NCH, CH), jnp.int32),
            pltpu.VMEM((EW,), jnp.float32),
            pltpu.VMEM((CH, CLUS), jnp.float32),
            pltpu.VMEM((CH, CLUS), jnp.float32),
            pltpu.VMEM((CH, CLUS), jnp.float32),
            pltpu.VMEM((CH, CLUS), jnp.float32),
            pltpu.VMEM((16,), jnp.float32),
            pltpu.SemaphoreType.DMA,
            pltpu.SemaphoreType.DMA,
            pltpu.SemaphoreType.DMA,
            pltpu.SemaphoreType.DMA,
        ],
    )
    return pass1, pass2


_RB = 2000
_NB = N_NODES // _RB


def _tc_body(agg_ref, deg_ref, x_ref, wa_ref, wx_ref, bcat_ref,
             w2e_w_ref, w2e_root_ref, w2e_b_ref,
             lin1_w_ref, lin1_b_ref, lin2_w_ref, lin2_b_ref,
             q_ref, sums2_ref, logp_ref,
             out_acc, s2_acc):
    i = pl.program_id(0)

    @pl.when(i == 0)
    def _init():
        out_acc[...] = jnp.zeros_like(out_acc)
        s2_acc[0] = 0.0

    ag = agg_ref[0] + agg_ref[1]
    cnt = jnp.maximum(ag[:, 5:6], 1.0)
    aggm = ag[:, 0:8] / cnt
    x = x_ref[...]
    z = (jnp.dot(aggm, wa_ref[...], preferred_element_type=jnp.float32)
         + jnp.dot(x, wx_ref[...], preferred_element_type=jnp.float32)
         + bcat_ref[...])
    mat_s = z[:, :CLUS]
    mat_y = jnp.maximum(z[:, CLUS:], 0.0)
    m = jnp.max(mat_s, axis=1, keepdims=True)
    e = jnp.exp(mat_s - m)
    s = jnp.sum(e, axis=1, keepdims=True)
    S = e / s
    deg = deg_ref[0][:, 0:1] + deg_ref[1][:, 0:1]
    dinv = jnp.where(deg > 0, lax.rsqrt(jnp.maximum(deg, 1e-12)), 0.0)
    q_ref[...] = dinv * S
    s2_acc[0] += jnp.sum(S * S)
    out_acc[...] += lax.dot_general(S, mat_y, (((0,), (0,)), ((), ())),
                                    preferred_element_type=jnp.float32)

    @pl.when(i == _NB - 1)
    def _head():
        out = out_acc[...]
        agg2 = jnp.dot(jnp.sum(out, axis=0, keepdims=True) / CLUS,
                       w2e_w_ref[...], preferred_element_type=jnp.float32)
        mat_y2 = jnp.maximum(
            agg2 + jnp.dot(out, w2e_root_ref[...],
                           preferred_element_type=jnp.float32)
            + w2e_b_ref[...], 0.0)
        out2 = jnp.sum(mat_y2, axis=0, keepdims=True)
        h = jnp.maximum(
            jnp.dot(out2, lin1_w_ref[...], preferred_element_type=jnp.float32)
            + lin1_b_ref[...], 0.0)
        h2 = (jnp.dot(h, lin2_w_ref[...], preferred_element_type=jnp.float32)
              + lin2_b_ref[...])
        mh = jnp.max(h2, axis=1, keepdims=True)
        lse = mh + jnp.log(jnp.sum(jnp.exp(h2 - mh), axis=1, keepdims=True))
        logp_ref[...] = h2 - lse
        sums2_ref[0, 0] = s2_acc[0]


def _tc_main(agg, deg, x, wa, wx, bcat, w2e_w, w2e_root, w2e_b,
             lin1_w, lin1_b, lin2_w, lin2_b):
    full = lambda *shape: pl.BlockSpec(shape, lambda i: (0,) * len(shape))
    return pl.pallas_call(
        _tc_body,
        grid=(_NB,),
        in_specs=[
            pl.BlockSpec((NC, _RB, 16), lambda i: (0, i, 0)),
            pl.BlockSpec((NC, _RB, 16), lambda i: (0, i, 0)),
            pl.BlockSpec((_RB, 5), lambda i: (i, 0)),
            full(8, 192), full(5, 192), full(1, 192),
            full(FOU1, FOU1), full(FOU1, FOU1), full(1, FOU1),
            full(FOU1, 256), full(1, 256), full(256, 8), full(1, 8),
        ],
        out_specs=[
            pl.BlockSpec((_RB, CLUS), lambda i: (i, 0)),
            pl.BlockSpec(memory_space=pltpu.SMEM),
            pl.BlockSpec((1, 8), lambda i: (0, 0)),
        ],
        out_shape=[
            jax.ShapeDtypeStruct((N_NODES, CLUS), jnp.float32),
            jax.ShapeDtypeStruct((1, 1), jnp.float32),
            jax.ShapeDtypeStruct((1, 8), jnp.float32),
        ],
        scratch_shapes=[
            pltpu.VMEM((CLUS, FOU1), jnp.float32),
            pltpu.SMEM((1,), jnp.float32),
        ],
    )(agg, deg, x, wa, wx, bcat, w2e_w, w2e_root, w2e_b,
      lin1_w, lin1_b, lin2_w, lin2_b)


def kernel(x, edge_index, edge_wht,
           w1p_w, w1p_root, w1p_b, w1e_w, w1e_root, w1e_b,
           w2p_w, w2p_root, w2p_b, w2e_w, w2e_root, w2e_b,
           lin1_w, lin1_b, lin2_w, lin2_b):
    f32 = jnp.float32
    src = edge_index[0].astype(jnp.int32)
    dst = edge_index[1].astype(jnp.int32)
    w = edge_wht.reshape(-1).astype(f32)

    src_r = src.reshape(NW, NCH, CH)
    dst_r = dst.reshape(NW, NCH, CH)
    w_r = w.reshape(NW, EW)
    xflat = x.reshape(-1).astype(f32)
    z8 = jnp.zeros((ROWS_PER_TILE, 16), f32)
    z4 = jnp.zeros((ROWS_PER_TILE, 16), f32)
    stag0 = jnp.zeros((CH, 16), f32).at[:, 5].set(1.0)

    wa = jnp.zeros((8, 192), f32).at[0:5, 0:128].set(w1p_w).at[0:5, 128:].set(w1e_w)
    wx = jnp.concatenate([w1p_root, w1e_root], axis=1)
    bcat = jnp.concatenate([w1p_b, w1e_b]).reshape(1, 192)

    sc_pass1, sc_pass2 = _sc_kernels()
    agg, deg = sc_pass1(xflat, src_r, dst_r, w_r, z8, z4, stag0)
    q, sums2, logp = _tc_main(
        agg, deg, x, wa, wx, bcat,
        w2e_w, w2e_root, w2e_b.reshape(1, FOU1),
        lin1_w, lin1_b.reshape(1, 256), lin2_w, lin2_b.reshape(1, 8))
    parts = sc_pass2(q, src_r, dst_r, w_r)
    reg1 = (sums2[0, 0] - jnp.sum(parts)) / f32(N_NODES)
    return logp, reg1
